# baseline (speedup 1.0000x reference)
import functools

import jax
import jax.numpy as jnp
from jax import lax
from jax.experimental import pallas as pl
from jax.experimental.pallas import tpu as pltpu
from jax.experimental.pallas import tpu_sc as plsc

N = 50000
T = 64
DT = 86400.0
NSUB = 4
OUTLET = N - 1

NS = 16
LANES = 16
CHUNK = 3200
NPAD = NS * CHUNK
NDUMP = 8
IDXW = 128
KROWS = CHUNK // IDXW
NSEG = 5
RPS = KROWS // NSEG
SEGW = RPS * IDXW
SZ = NPAD + NDUMP
NVREG = CHUNK // LANES
DT_SUB = DT / NSUB

OUT_SUBCORE = OUTLET // CHUNK
OUT_LOCAL = OUTLET - OUT_SUBCORE * CHUNK
OUT_VREG = OUT_LOCAL // LANES
OUT_LANE = OUT_LOCAL % LANES

_LN2_HI = 0.693359375
_LN2_LO = -2.12194440e-4
_SQRT2 = 1.41421356237
_C27 = 0.27 ** (2.0 / 3.0)


def _ln16(x):
    bits = plsc.bitcast(x, jnp.int32)
    e = lax.shift_right_arithmetic(bits - 0x3F2AAAAB, 23)
    m = plsc.bitcast(bits - lax.shift_left(e, 23), jnp.float32)
    ef = e.astype(jnp.float32)
    f = m - 1.0
    z = f * f
    y = jnp.float32(0.16151336)
    y = y * f - 0.18353264
    y = y * f + 0.19928537
    y = y * f - 0.24958651
    y = y * f + 0.3333372
    y = f * z * y
    y = y + ef * _LN2_LO
    y = y - 0.5 * z
    return f + y + ef * _LN2_HI


_mesh = plsc.VectorSubcoreMesh(core_axis_name="c", subcore_axis_name="s")


@functools.partial(
    pl.kernel,
    out_type=jax.ShapeDtypeStruct((T * LANES,), jnp.float32),
    mesh=_mesh,
    compiler_params=pltpu.CompilerParams(needs_layout_passes=False),
    scratch_types=[
        pltpu.VMEM_SHARED((2 * SZ,), jnp.float32),
        pltpu.VMEM((CHUNK,), jnp.float32),
        pltpu.VMEM((CHUNK,), jnp.float32),
        pltpu.VMEM((CHUNK,), jnp.float32),
        pltpu.VMEM((2 * CHUNK,), jnp.float32),
        pltpu.VMEM((CHUNK,), jnp.float32),
        pltpu.VMEM((CHUNK,), jnp.float32),
        pltpu.VMEM((CHUNK,), jnp.float32),
        pltpu.VMEM((CHUNK,), jnp.float32),
        pltpu.VMEM((2, KROWS, IDXW), jnp.int32),
        pltpu.VMEM((T * LANES,), jnp.float32),
        pltpu.SemaphoreType.DMA,
        pltpu.SemaphoreType.DMA,
        pltpu.SemaphoreType.DMA,
        pltpu.SemaphoreType.DMA,
    ],
)
def _routing_kernel(
    lat_hbm, logn_hbm, len_hbm, slope_hbm, dst_hbm, out_hbm,
    shared_up, q_v, in_v, up_v, lat_v, celc_v, xc_v, len_v, zero_v,
    idx_v, out_v, scat_sem, zero_sem, read_sem, lat_sem,
):
    cid = lax.axis_index("c")
    sid = lax.axis_index("s")
    base = sid * CHUNK

    pltpu.sync_copy(dst_hbm.at[sid], idx_v)
    pltpu.sync_copy(len_hbm.at[pl.ds(base, CHUNK)], len_v)
    pltpu.sync_copy(logn_hbm.at[pl.ds(base, CHUNK)], up_v)
    pltpu.sync_copy(
        slope_hbm.at[pl.ds(base, CHUNK)],
        lat_v.at[pl.ds(0, CHUNK)])

    @plsc.parallel_loop(0, CHUNK, step=LANES)
    def _init(off):
        ds = pl.ds(off, LANES)
        ln_n = up_v[ds]
        sl = lat_v[ds]
        ln = len_v[ds]
        celc_v[ds] = (5.0 / 3.0) * _C27 * jnp.exp(0.5 * _ln16(sl) - ln_n)
        xc_v[ds] = 1.0 / (14.4 * sl * ln)
        len_v[ds] = 2.0 * ln
        ones = jnp.full((LANES,), 1.0, jnp.float32)
        q_v[ds] = ones
        in_v[ds] = ones
        zero_v[ds] = jnp.zeros((LANES,), jnp.float32)

    pltpu.sync_copy(zero_v, shared_up.at[pl.ds(base, CHUNK)])
    pltpu.async_copy(
        zero_v, shared_up.at[pl.ds(SZ + base, CHUNK)], zero_sem)
    pltpu.async_copy(
        lat_hbm.at[0, pl.ds(base, CHUNK)], lat_v.at[pl.ds(0, CHUNK)],
        lat_sem)
    plsc.subcore_barrier()

    def fire_row(bb, k):
        pltpu.async_copy(
            q_v.at[pl.ds(k * IDXW, IDXW)],
            shared_up.at[idx_v.at[bb, k]],
            scat_sem,
            add=True,
        )

    def prol_fire(k, _):
        fire_row(0, k)
        return 0

    lax.fori_loop(0, KROWS, prol_fire, 0)

    def sem_drain(sem):
        pltpu.make_async_copy(len_hbm.at[pl.ds(0, CHUNK)], up_v, sem).wait()

    def substep(s, latoff):
        b = jnp.bitwise_and(s, 1)
        bb = 1 - b
        off = b * SZ + base

        sem_drain(scat_sem)
        sem_drain(zero_sem)
        plsc.subcore_barrier()

        pltpu.sync_copy(
            shared_up.at[pl.ds(off, SEGW)], up_v.at[pl.ds(0, SEGW)])
        rest_read = pltpu.async_copy(
            shared_up.at[pl.ds(off + SEGW, CHUNK - SEGW)],
            up_v.at[pl.ds(SEGW, CHUNK - SEGW)],
            read_sem,
        )

        for seg in range(NSEG):
            if seg == 1:
                rest_read.wait()
                pltpu.async_copy(
                    zero_v, shared_up.at[pl.ds(off, CHUNK)], zero_sem)

            @plsc.parallel_loop(
                seg * RPS * IDXW, (seg + 1) * RPS * IDXW, step=LANES,
                unroll=8)
            def _mc(off):
                ds = pl.ds(off, LANES)
                q = q_v[ds]
                in_prev = in_v[ds]
                inflow = lat_v[pl.ds(latoff + off, LANES)] + up_v[ds]
                qref = jnp.maximum(0.5 * (inflow + q), 1e-6)
                p01 = jnp.exp(0.1 * _ln16(qref))
                p02 = p01 * p01
                p04 = p02 * p02
                sqrtq = p04 * p01
                cel = jnp.maximum(celc_v[ds] * p02, 1e-4)
                hc = 0.5 * cel
                cx = jnp.clip(hc - xc_v[ds] * sqrtq, 0.0, hc)
                cel2 = cel * cel
                s1 = in_prev - inflow
                s2 = inflow + in_prev - 2.0 * q
                num = (len_v[ds] * cx) * s1 + DT_SUB * (cel2 * s2)
                den = len_v[ds] * (cel - cx) + DT_SUB * cel2
                q_v[ds] = jnp.maximum(num / den + q, 0.0)
                in_v[ds] = inflow

            def fire_seg(k, _):
                fire_row(bb, k)
                return 0

            lax.fori_loop(seg * RPS, (seg + 1) * RPS, fire_seg, 0)
        return latoff

    def timestep(t, _):
        sem_drain(lat_sem)
        latoff = jnp.bitwise_and(t, 1) * CHUNK
        nxt = jnp.minimum(t + 1, T - 1)
        pltpu.async_copy(
            lat_hbm.at[nxt, pl.ds(base, CHUNK)],
            lat_v.at[pl.ds(CHUNK - latoff, CHUNK)],
            lat_sem,
        )
        lax.fori_loop(0, NSUB, substep, latoff)
        out_v[pl.ds(t * LANES, LANES)] = q_v[pl.ds(OUT_VREG * LANES, LANES)]
        return 0

    lax.fori_loop(0, T, timestep, 0)

    sem_drain(scat_sem)
    sem_drain(zero_sem)
    sem_drain(lat_sem)

    @pl.when(jnp.logical_and(cid == 0, sid == OUT_SUBCORE))
    def _():
        pltpu.sync_copy(out_v, out_hbm)


def kernel(lateral_inflow, log_manning_n, length, slope, downstream_idx):
    pad = NPAD - N
    lat = jnp.pad(lateral_inflow, ((0, 0), (0, pad)))
    logn = jnp.pad(log_manning_n, (0, pad))
    leng = jnp.pad(length, (0, pad), constant_values=1000.0)
    slp = jnp.pad(slope, (0, pad), constant_values=0.01)
    pad_idx = NPAD + (jnp.arange(pad, dtype=jnp.int32) % NDUMP)
    dst = jnp.concatenate([downstream_idx.astype(jnp.int32), pad_idx])
    dst = jnp.stack([dst, dst + SZ], axis=0)
    dst = dst.reshape(2, NS, KROWS, IDXW).transpose(1, 0, 2, 3)
    out = _routing_kernel(lat, logn, leng, slp, dst)
    return out.reshape(T, LANES)[:, OUT_LANE]


# speedup vs baseline: 1.4360x; 1.4360x over previous
import functools

import jax
import jax.numpy as jnp
from jax import lax
from jax.experimental import pallas as pl
from jax.experimental.pallas import tpu as pltpu
from jax.experimental.pallas import tpu_sc as plsc

N = 50000
T = 64
DT = 86400.0
NSUB = 4
OUTLET = N - 1

NS = 16
LANES = 16
CHUNK = 3200
NPAD = NS * CHUNK
NDUMP = 8
IDXW = 128
KROWS = CHUNK // IDXW
NSEG = 5
RPS = KROWS // NSEG
SEGW = RPS * IDXW
SZ = NPAD + NDUMP
NVREG = CHUNK // LANES
DT_SUB = DT / NSUB

OUT_SUBCORE = OUTLET // CHUNK
OUT_LOCAL = OUTLET - OUT_SUBCORE * CHUNK
OUT_VREG = OUT_LOCAL // LANES
OUT_LANE = OUT_LOCAL % LANES

_LN2_HI = 0.693359375
_LN2_LO = -2.12194440e-4
_SQRT2 = 1.41421356237
_C27 = 0.27 ** (2.0 / 3.0)


def _ln16(x):
    bits = plsc.bitcast(x, jnp.int32)
    e = lax.shift_right_arithmetic(bits - 0x3F2AAAAB, 23)
    m = plsc.bitcast(bits - lax.shift_left(e, 23), jnp.float32)
    ef = e.astype(jnp.float32)
    f = m - 1.0
    z = f * f
    y = jnp.float32(0.16151336)
    y = y * f - 0.18353264
    y = y * f + 0.19928537
    y = y * f - 0.24958651
    y = y * f + 0.3333372
    y = f * z * y
    y = y + ef * _LN2_LO
    y = y - 0.5 * z
    return f + y + ef * _LN2_HI


_mesh = plsc.VectorSubcoreMesh(core_axis_name="c", subcore_axis_name="s")


@functools.partial(
    pl.kernel,
    out_type=jax.ShapeDtypeStruct((T * LANES,), jnp.float32),
    mesh=_mesh,
    compiler_params=pltpu.CompilerParams(needs_layout_passes=False),
    scratch_types=[
        pltpu.VMEM_SHARED((2 * SZ,), jnp.float32),
        pltpu.VMEM((CHUNK,), jnp.float32),
        pltpu.VMEM((CHUNK,), jnp.float32),
        pltpu.VMEM((CHUNK,), jnp.float32),
        pltpu.VMEM((2 * CHUNK,), jnp.float32),
        pltpu.VMEM((CHUNK,), jnp.float32),
        pltpu.VMEM((CHUNK,), jnp.float32),
        pltpu.VMEM((CHUNK,), jnp.float32),
        pltpu.VMEM((2, KROWS, IDXW), jnp.int32),
        pltpu.VMEM((T * LANES,), jnp.float32),
        pltpu.SemaphoreType.DMA,
        pltpu.SemaphoreType.DMA,
        pltpu.SemaphoreType.DMA,
        pltpu.SemaphoreType.DMA,
    ],
)
def _routing_kernel(
    lat_hbm, logn_hbm, len_hbm, slope_hbm, dst_hbm, out_hbm,
    shared_up, q_v, in_v, up_v, lat_v, celc_v, xc_v, len_v,
    idx_v, out_v, scat_sem, zero_sem, read_sem, lat_sem,
):
    cid = lax.axis_index("c")
    sid = lax.axis_index("s")
    base = sid * CHUNK

    pltpu.sync_copy(dst_hbm.at[sid], idx_v)
    pltpu.sync_copy(len_hbm.at[pl.ds(base, CHUNK)], len_v)
    pltpu.sync_copy(logn_hbm.at[pl.ds(base, CHUNK)], up_v)
    pltpu.sync_copy(
        slope_hbm.at[pl.ds(base, CHUNK)],
        lat_v.at[pl.ds(0, CHUNK)])

    @plsc.parallel_loop(0, CHUNK, step=LANES)
    def _init(off):
        ds = pl.ds(off, LANES)
        ln_n = up_v[ds]
        sl = lat_v[ds]
        ln = len_v[ds]
        celc_v[ds] = (5.0 / 3.0) * _C27 * jnp.exp(0.5 * _ln16(sl) - ln_n)
        xc_v[ds] = 1.0 / (14.4 * sl * ln)
        len_v[ds] = 2.0 * ln
        ones = jnp.full((LANES,), 1.0, jnp.float32)
        q_v[ds] = ones
        in_v[ds] = ones

    def sem_drain(sem):
        pltpu.make_async_copy(len_hbm.at[pl.ds(0, CHUNK)], up_v, sem).wait()

    pltpu.async_copy(
        lat_hbm.at[0, pl.ds(base, CHUNK)], lat_v.at[pl.ds(0, CHUNK)],
        lat_sem)
    sem_drain(lat_sem)
    pltpu.sync_copy(
        lat_v.at[pl.ds(0, CHUNK)], shared_up.at[pl.ds(base, CHUNK)])
    pltpu.async_copy(
        lat_v.at[pl.ds(0, CHUNK)], shared_up.at[pl.ds(SZ + base, CHUNK)],
        zero_sem)
    plsc.subcore_barrier()

    def fire_row(bb, k):
        pltpu.async_copy(
            q_v.at[pl.ds(k * IDXW, IDXW)],
            shared_up.at[idx_v.at[bb, k]],
            scat_sem,
            add=True,
        )

    def prol_fire(k, _):
        fire_row(0, k)
        return 0

    lax.fori_loop(0, KROWS, prol_fire, 0)

    def substep(s, init_off):
        b = jnp.bitwise_and(s, 1)
        bb = 1 - b
        off = b * SZ + base

        sem_drain(scat_sem)
        sem_drain(zero_sem)
        plsc.subcore_barrier()

        pltpu.sync_copy(
            shared_up.at[pl.ds(off, SEGW)], up_v.at[pl.ds(0, SEGW)])
        rest_read = pltpu.async_copy(
            shared_up.at[pl.ds(off + SEGW, CHUNK - SEGW)],
            up_v.at[pl.ds(SEGW, CHUNK - SEGW)],
            read_sem,
        )

        for seg in range(NSEG):
            if seg == 1:
                rest_read.wait()
                pltpu.async_copy(
                    lat_v.at[pl.ds(init_off, CHUNK)],
                    shared_up.at[pl.ds(off, CHUNK)], zero_sem)

            @plsc.parallel_loop(
                seg * RPS * IDXW, (seg + 1) * RPS * IDXW, step=LANES,
                unroll=4)
            def _mc(off):
                ds = pl.ds(off, LANES)
                q = q_v[ds]
                in_prev = in_v[ds]
                inflow = up_v[ds]
                qref = jnp.maximum(0.5 * (inflow + q), 1e-6)
                p01 = jnp.exp(0.1 * _ln16(qref))
                p02 = p01 * p01
                p04 = p02 * p02
                sqrtq = p04 * p01
                cel = jnp.maximum(celc_v[ds] * p02, 1e-4)
                hc = 0.5 * cel
                cx = jnp.clip(hc - xc_v[ds] * sqrtq, 0.0, hc)
                cel2 = cel * cel
                s1 = in_prev - inflow
                s2 = inflow + in_prev - 2.0 * q
                num = (len_v[ds] * cx) * s1 + DT_SUB * (cel2 * s2)
                den = len_v[ds] * (cel - cx) + DT_SUB * cel2
                q_v[ds] = jnp.maximum(num / den + q, 0.0)
                in_v[ds] = inflow

            def fire_seg(k, _):
                fire_row(bb, k)
                return 0

            lax.fori_loop(seg * RPS, (seg + 1) * RPS, fire_seg, 0)
        return init_off

    def timestep(t, _):
        latoff = jnp.bitwise_and(t, 1) * CHUNK
        nxt = jnp.minimum(t + 1, T - 1)
        pltpu.async_copy(
            lat_hbm.at[nxt, pl.ds(base, CHUNK)],
            lat_v.at[pl.ds(CHUNK - latoff, CHUNK)],
            lat_sem,
        )
        lax.fori_loop(0, 2, substep, latoff)
        sem_drain(lat_sem)
        lax.fori_loop(2, NSUB, substep, CHUNK - latoff)
        out_v[pl.ds(t * LANES, LANES)] = q_v[pl.ds(OUT_VREG * LANES, LANES)]
        return 0

    lax.fori_loop(0, T, timestep, 0)

    sem_drain(scat_sem)
    sem_drain(zero_sem)

    @pl.when(jnp.logical_and(cid == 0, sid == OUT_SUBCORE))
    def _():
        pltpu.sync_copy(out_v, out_hbm)


def kernel(lateral_inflow, log_manning_n, length, slope, downstream_idx):
    pad = NPAD - N
    lat = jnp.pad(lateral_inflow, ((0, 0), (0, pad)))
    logn = jnp.pad(log_manning_n, (0, pad))
    leng = jnp.pad(length, (0, pad), constant_values=1000.0)
    slp = jnp.pad(slope, (0, pad), constant_values=0.01)
    pad_idx = NPAD + (jnp.arange(pad, dtype=jnp.int32) % NDUMP)
    dst = jnp.concatenate([downstream_idx.astype(jnp.int32), pad_idx])
    dst = jnp.stack([dst, dst + SZ], axis=0)
    dst = dst.reshape(2, NS, KROWS, IDXW).transpose(1, 0, 2, 3)
    out = _routing_kernel(lat, logn, leng, slp, dst)
    return out.reshape(T, LANES)[:, OUT_LANE]
